# Initial kernel scaffold; baseline (speedup 1.0000x reference)
import jax
import jax.numpy as jnp
from jax.experimental import pallas as pl


def kernel(node_feats, edge_index, edge_geo, cond, batch_ids, W_np, b_np, g_np, be_np, W_geo, b_geo, g_geo, be_geo, Wc1, bc1, g_c, be_c, Wc2, bc2, Wf, bf):
    raise NotImplementedError("write your pallas kernel here")


# speedup vs baseline: 4.0210x; 4.0210x over previous
import functools

import jax
import jax.numpy as jnp
from jax import lax
from jax.experimental import pallas as pl
from jax.experimental.pallas import tpu as pltpu
from jax.experimental.pallas import tpu_sc as plsc

F32 = jnp.float32
EPS = 1e-5

NC, NS = 2, 16
NW = NC * NS


def _node_proj_body(x_ref, w_ref, wf1_ref, v_ref, o_ref):
    h = jnp.dot(x_ref[...], w_ref[...], preferred_element_type=F32)
    h = h + v_ref[0:1, :]
    m = jnp.mean(h, axis=-1, keepdims=True)
    hc = h - m
    var = jnp.mean(hc * hc, axis=-1, keepdims=True)
    y = hc * lax.rsqrt(var + EPS) * v_ref[1:2, :] + v_ref[2:3, :]
    o_ref[...] = jnp.dot(y, wf1_ref[...], preferred_element_type=F32)


def _node_proj(node_feats, W_np, Wf1, b_np, g_np, be_np):
    n, k = node_feats.shape
    blk = 2000
    grid = n // blk
    vecs = jnp.concatenate(
        [b_np[None], g_np[None], be_np[None], jnp.zeros((5, 128), F32)], axis=0
    )
    return pl.pallas_call(
        _node_proj_body,
        grid=(grid,),
        in_specs=[
            pl.BlockSpec((blk, k), lambda i: (i, 0)),
            pl.BlockSpec((k, 128), lambda i: (0, 0)),
            pl.BlockSpec((128, 128), lambda i: (0, 0)),
            pl.BlockSpec((8, 128), lambda i: (0, 0)),
        ],
        out_specs=pl.BlockSpec((blk, 128), lambda i: (i, 0)),
        out_shape=jax.ShapeDtypeStruct((n, 128), F32),
    )(node_feats, W_np, Wf1, vecs)


def _cond_proj_body(c_ref, w1_ref, w2_ref, v_ref, b2_ref, o_ref):
    h = jnp.dot(c_ref[...], w1_ref[...], preferred_element_type=F32)
    h = h + v_ref[0:1, :]
    m = jnp.mean(h, axis=-1, keepdims=True)
    hc = h - m
    var = jnp.mean(hc * hc, axis=-1, keepdims=True)
    y = hc * lax.rsqrt(var + EPS) * v_ref[1:2, :] + v_ref[2:3, :]
    gb = jnp.dot(y, w2_ref[...], preferred_element_type=F32) + b2_ref[0:1, :]
    lane = lax.broadcasted_iota(jnp.int32, gb.shape, 1)
    o_ref[...] = gb + (lane < 128).astype(F32)


def _cond_proj(cond, Wc1, bc1, g_c, be_c, Wc2, bc2):
    vecs = jnp.concatenate(
        [bc1[None], g_c[None], be_c[None], jnp.zeros((5, 128), F32)], axis=0
    )
    b2 = jnp.concatenate([bc2[None], jnp.zeros((7, 256), F32)], axis=0)
    return pl.pallas_call(
        _cond_proj_body,
        grid=(1,),
        in_specs=[
            pl.BlockSpec((16, 128), lambda i: (0, 0)),
            pl.BlockSpec((128, 128), lambda i: (0, 0)),
            pl.BlockSpec((128, 256), lambda i: (0, 0)),
            pl.BlockSpec((8, 128), lambda i: (0, 0)),
            pl.BlockSpec((8, 256), lambda i: (0, 0)),
        ],
        out_specs=pl.BlockSpec((16, 256), lambda i: (0, 0)),
        out_shape=jax.ShapeDtypeStruct((16, 256), F32),
    )(cond, Wc1, Wc2, vecs, b2)


def _sc_gather_sum(nf2, src, dst, n_edges):
    C = 128
    n_groups = n_edges // C
    base_g, extra = divmod(n_groups, NW)

    mesh = plsc.VectorSubcoreMesh(
        core_axis_name="c", subcore_axis_name="s", num_cores=NC, num_subcores=NS
    )

    @functools.partial(
        pl.kernel,
        out_type=jax.ShapeDtypeStruct((n_edges, 128), F32),
        mesh=mesh,
        scratch_types=[
            pltpu.VMEM((C,), jnp.int32),
            pltpu.VMEM((C,), jnp.int32),
            pltpu.VMEM((C, 128), F32),
            pltpu.VMEM((C, 128), F32),
            pltpu.SemaphoreType.DMA,
            pltpu.SemaphoreType.DMA,
        ],
    )
    def sc_kernel(nf2_hbm, src_hbm, dst_hbm, out_hbm, idx_s, idx_d, rows_a,
                  rows_b, sem_a, sem_b):
        wid = lax.axis_index("s") * NC + lax.axis_index("c")
        g0 = wid * base_g + jnp.minimum(wid, extra)
        ng = base_g + jnp.where(wid < extra, 1, 0)

        def group_body(i, carry):
            base = (g0 + i) * C
            pltpu.sync_copy(src_hbm.at[pl.ds(base, C)], idx_s)
            pltpu.sync_copy(dst_hbm.at[pl.ds(base, C)], idx_d)
            cp_a = pltpu.async_copy(nf2_hbm.at[idx_s], rows_a, sem_a)
            cp_b = pltpu.async_copy(nf2_hbm.at[idx_d], rows_b, sem_b)
            cp_a.wait()
            cp_b.wait()

            def add_body(r, c2):
                for cc in range(8):
                    sl = pl.ds(cc * 16, 16)
                    rows_a[r, sl] = rows_a[r, sl] + rows_b[r, sl]
                return c2

            lax.fori_loop(0, C, add_body, 0)
            pltpu.sync_copy(rows_a, out_hbm.at[pl.ds(base, C)])
            return carry

        lax.fori_loop(0, ng, group_body, 0)

    return sc_kernel(nf2, src, dst)


def _edge_body(s_ref, x_ref, bid_ref, a_ref, cv_ref, sm_ref, gbt_ref, o_ref):
    x = x_ref[...]
    s_in = s_ref[...]
    c1 = cv_ref[0:1, :]
    vrow = cv_ref[1:2, :]
    crow = cv_ref[2:3, :]
    bbar = cv_ref[3, 0]
    ccst = cv_ref[3, 1]
    mc = sm_ref[0:8, :]
    m8 = sm_ref[8:9, :]
    uc = sm_ref[9:10, :]

    t = jnp.dot(x, a_ref[...], preferred_element_type=F32) + c1
    mu = jnp.sum(x * m8, axis=-1, keepdims=True) + bbar
    xm = jnp.dot(x, mc, preferred_element_type=F32)
    varg = (jnp.sum(xm * x, axis=-1, keepdims=True)
            + 2.0 * jnp.sum(x * uc, axis=-1, keepdims=True) + ccst)
    inv_sg = lax.rsqrt(varg + EPS)

    y0 = s_in + (t - mu * vrow) * inv_sg + crow
    m = jnp.mean(y0, axis=-1, keepdims=True)
    yc = y0 - m
    var = jnp.mean(yc * yc, axis=-1, keepdims=True)
    y = yc * lax.rsqrt(var + EPS)

    oh = (bid_ref[...] == lax.broadcasted_iota(jnp.int32, (y.shape[0], 16), 1))
    gb = jnp.dot(oh.astype(F32), gbt_ref[...], preferred_element_type=F32)
    o_ref[...] = jnp.maximum(y * gb[:, :128] + gb[:, 128:], 0.0)


def _edge_fuse(S, edge_geo, bid2d, A, cvec, small8, gbt):
    n_edges = S.shape[0]
    blk = 1280
    grid = n_edges // blk
    return pl.pallas_call(
        _edge_body,
        grid=(grid,),
        in_specs=[
            pl.BlockSpec((blk, 128), lambda i: (i, 0)),
            pl.BlockSpec((blk, 8), lambda i: (i, 0)),
            pl.BlockSpec((blk, 1), lambda i: (i, 0)),
            pl.BlockSpec((8, 128), lambda i: (0, 0)),
            pl.BlockSpec((8, 128), lambda i: (0, 0)),
            pl.BlockSpec((16, 8), lambda i: (0, 0)),
            pl.BlockSpec((16, 256), lambda i: (0, 0)),
        ],
        out_specs=pl.BlockSpec((blk, 128), lambda i: (i, 0)),
        out_shape=jax.ShapeDtypeStruct((n_edges, 128), F32),
    )(S, edge_geo, bid2d, A, cvec, small8, gbt)


def kernel(node_feats, edge_index, edge_geo, cond, batch_ids,
           W_np, b_np, g_np, be_np,
           W_geo, b_geo, g_geo, be_geo,
           Wc1, bc1, g_c, be_c, Wc2, bc2,
           Wf, bf):
    n_edges = edge_index.shape[1]
    src = edge_index[0].astype(jnp.int32)
    dst = edge_index[1].astype(jnp.int32)
    bid2d = batch_ids.astype(jnp.int32).reshape(n_edges, 1)

    Wf1 = Wf[:128]
    Wf2 = Wf[128:]

    Wgsum = W_geo.reshape(8, 8, 128).sum(axis=0)
    A = Wgsum @ (g_geo[:, None] * Wf2)
    c1 = (b_geo * g_geo) @ Wf2
    vrow = g_geo @ Wf2
    crow = be_geo @ Wf2 + bf
    m8 = Wgsum.mean(axis=1)
    bbar = b_geo.mean()
    acen = Wgsum - m8[:, None]
    bcen = b_geo - bbar
    Mc = (acen @ acen.T) / 128.0
    uc = (acen @ bcen) / 128.0
    ccst = jnp.dot(bcen, bcen) / 128.0

    misc = jnp.zeros((128,), F32).at[0].set(bbar).at[1].set(ccst)
    cvec = jnp.concatenate(
        [c1[None], vrow[None], crow[None], misc[None], jnp.zeros((4, 128), F32)],
        axis=0,
    )
    small8 = jnp.concatenate(
        [Mc, m8[None], uc[None], jnp.zeros((6, 8), F32)], axis=0
    )

    nf2 = _node_proj(node_feats, W_np, Wf1, b_np, g_np, be_np)
    gbt = _cond_proj(cond, Wc1, bc1, g_c, be_c, Wc2, bc2)
    S = _sc_gather_sum(nf2, src, dst, n_edges)
    return _edge_fuse(S, edge_geo, bid2d, A, cvec, small8, gbt)
